# Initial kernel scaffold; baseline (speedup 1.0000x reference)
import jax
import jax.numpy as jnp
from jax.experimental import pallas as pl


def kernel(hidden, edges, n_node, old_nodes_new_idx, rela_embed, Ws, Wr, w_alpha_w, w_alpha_b, W_h):
    raise NotImplementedError("write your pallas kernel here")


# speedup vs baseline: 2.2256x; 2.2256x over previous
import functools

import jax
import jax.numpy as jnp
from jax import lax
from jax.experimental import pallas as pl
from jax.experimental.pallas import tpu as pltpu
from jax.experimental.pallas import tpu_sc as plsc

NC = 2
NS = 16
NW = NC * NS
C = 80
L = 16


def _matmul_body(x_ref, w_ref, o_ref):
    o_ref[...] = jnp.dot(x_ref[...], w_ref[...],
                         preferred_element_type=jnp.float32)


def _matmul(x, w, br):
    rows, d = x.shape
    return pl.pallas_call(
        _matmul_body,
        grid=(rows // br,),
        in_specs=[
            pl.BlockSpec((br, d), lambda i: (i, 0)),
            pl.BlockSpec((d, w.shape[1]), lambda i: (0, 0)),
        ],
        out_specs=pl.BlockSpec((br, w.shape[1]), lambda i: (i, 0)),
        out_shape=jax.ShapeDtypeStruct((rows, w.shape[1]), jnp.float32),
    )(x, w)


def _final_body(p_ref, w_ref, o_ref):
    acc = p_ref[0] + p_ref[1]
    o_ref[...] = jnp.dot(acc, w_ref[...], preferred_element_type=jnp.float32)


def _final(p, w, br, rows):
    d = p.shape[2]
    return pl.pallas_call(
        _final_body,
        grid=(rows // br,),
        in_specs=[
            pl.BlockSpec((2, br, d), lambda i: (0, i, 0)),
            pl.BlockSpec((d, d), lambda i: (0, 0)),
        ],
        out_specs=pl.BlockSpec((br, d), lambda i: (i, 0)),
        out_shape=jax.ShapeDtypeStruct((rows, d), jnp.float32),
    )(p, w)


def _sc_edges(h_t, as_t, r_t, ar_t, sub_i, rel_i, obj_i, wab, n_node, npad):
    d = h_t.shape[1]
    e = sub_i.shape[0]
    e_per_w = e // NW
    n_chunks = e_per_w // C
    rows_per_tile = npad // NS
    dch = d // L

    mesh = plsc.VectorSubcoreMesh(core_axis_name="c", subcore_axis_name="s",
                                  num_cores=NC, num_subcores=NS)

    @functools.partial(
        pl.kernel,
        out_type=jax.ShapeDtypeStruct((NC, npad, d), jnp.float32),
        mesh=mesh,
        compiler_params=pltpu.CompilerParams(needs_layout_passes=False),
        scratch_types=[
            pltpu.VMEM((C,), jnp.int32),
            pltpu.VMEM((C,), jnp.int32),
            pltpu.VMEM((C,), jnp.int32),
            pltpu.VMEM((C, 128), jnp.float32),
            pltpu.VMEM((C, 128), jnp.float32),
            pltpu.VMEM((C, 128), jnp.float32),
            pltpu.VMEM((C,), jnp.float32),
            pltpu.VMEM((L * L,), jnp.float32),
            pltpu.VMEM((144,), jnp.float32),
            pltpu.VMEM_SHARED((npad, d), jnp.float32),
            pltpu.SemaphoreType.DMA,
            pltpu.SemaphoreType.DMA,
        ],
    )
    def k(h_hbm, as_hbm, r_hbm, ar_hbm, sub_hbm, rel_hbm, obj_hbm, wab_hbm,
          out_hbm, idx_sub, idx_rel, idx_obj, buf1, buf2, msg, albuf, sbuf,
          wvec, acc, sem1, sem2):
        cid = lax.axis_index("c")
        sid = lax.axis_index("s")
        wid = cid * NS + sid

        def zfill(r, _):
            for j in range(dch):
                msg[r, pl.ds(j * L, L)] = jnp.zeros((L,), jnp.float32)
            return 0

        lax.fori_loop(0, C, zfill, 0)
        for part in range(rows_per_tile // C):
            pltpu.sync_copy(
                msg, acc.at[pl.ds(sid * rows_per_tile + part * C, C)])

        pltpu.sync_copy(wab_hbm, wvec)
        wv = [wvec[pl.ds(j * L, L)] for j in range(dch)]
        bias_v = jnp.full((L,), wvec[pl.ds(d, L)][0], jnp.float32)
        ev16 = lax.iota(jnp.int32, L) * L

        plsc.subcore_barrier()

        base0 = wid * e_per_w

        def chunk_body(c, _):
            base = base0 + c * C
            pltpu.sync_copy(sub_hbm.at[pl.ds(base, C)], idx_sub)
            pltpu.sync_copy(rel_hbm.at[pl.ds(base, C)], idx_rel)
            pltpu.sync_copy(obj_hbm.at[pl.ds(base, C)], idx_obj)
            for i in range(C // L):
                v = idx_obj[pl.ds(i * L, L)]
                idx_obj[pl.ds(i * L, L)] = lax.rem(v, jnp.int32(n_node))

            cp1 = pltpu.async_copy(as_hbm.at[idx_sub], buf1, sem1)
            cp2 = pltpu.async_copy(ar_hbm.at[idx_rel], buf2, sem2)
            cp1.wait()
            cp2.wait()

            def alpha_body(g, _):
                ebase = g * L
                for e16 in range(L):
                    i = ebase + e16
                    s = jnp.zeros((L,), jnp.float32)
                    for j in range(dch):
                        a = buf1[i, pl.ds(j * L, L)] + buf2[i, pl.ds(j * L, L)]
                        s = s + jnp.maximum(a, 0.0) * wv[j]
                    sbuf[pl.ds(e16 * L, L)] = s
                tsum = jnp.zeros((L,), jnp.float32)
                for c16 in range(L):
                    tsum = tsum + plsc.load_gather(sbuf, [ev16 + c16])
                alpha_v = 1.0 / (1.0 + jnp.exp(-(tsum + bias_v)))
                albuf[pl.ds(ebase, L)] = alpha_v
                return 0

            lax.fori_loop(0, C // L, alpha_body, 0)

            cp1 = pltpu.async_copy(h_hbm.at[idx_sub], buf1, sem1)
            cp2 = pltpu.async_copy(r_hbm.at[idx_rel], buf2, sem2)
            cp1.wait()
            cp2.wait()

            def msg_body(g, _):
                ebase = g * L
                alpha_v = albuf[pl.ds(ebase, L)]
                for e16 in range(L):
                    i = ebase + e16
                    av = jnp.full((L,), alpha_v[e16], jnp.float32)
                    for j in range(dch):
                        m = buf1[i, pl.ds(j * L, L)] + buf2[i, pl.ds(j * L, L)]
                        msg[i, pl.ds(j * L, L)] = av * m
                return 0

            lax.fori_loop(0, C // L, msg_body, 0)
            pltpu.sync_copy(msg, acc.at[idx_obj], add=True)
            return 0

        lax.fori_loop(0, n_chunks, chunk_body, 0)

        plsc.subcore_barrier()
        pltpu.sync_copy(
            acc.at[pl.ds(sid * rows_per_tile, rows_per_tile)],
            out_hbm.at[cid, pl.ds(sid * rows_per_tile, rows_per_tile)])

    return k(h_t, as_t, r_t, ar_t, sub_i, rel_i, obj_i, wab)


def kernel(hidden, edges, n_node, old_nodes_new_idx, rela_embed, Ws, Wr,
           w_alpha_w, w_alpha_b, W_h):
    n, d = hidden.shape
    v = rela_embed.shape[0]

    sub_i = edges[:, 4]
    rel_i = edges[:, 2]
    obj_i = edges[:, 5]

    vpad = ((v + 399) // 400) * 400
    rela_p = jnp.pad(rela_embed, ((0, vpad - v), (0, 0)))

    as_t = _matmul(hidden, Ws, 400)
    ar_t = _matmul(rela_p, Wr, 400)

    wab = jnp.concatenate([
        w_alpha_w.reshape(d),
        w_alpha_b.reshape(1),
        jnp.zeros((15,), jnp.float32),
    ])

    npad = ((n + NS * 80 - 1) // (NS * 80)) * (NS * 80)
    partials = _sc_edges(hidden, as_t, rela_p, ar_t, sub_i, rel_i, obj_i,
                         wab, n, npad)
    return _final(partials, W_h, 400, n)


# speedup vs baseline: 3.1752x; 1.4267x over previous
import functools

import jax
import jax.numpy as jnp
from jax import lax
from jax.experimental import pallas as pl
from jax.experimental.pallas import tpu as pltpu
from jax.experimental.pallas import tpu_sc as plsc

NC = 2
NS = 16
NW = NC * NS
C = 80
L = 16


def _matmul_body(x_ref, w_ref, o_ref):
    o_ref[...] = jnp.dot(x_ref[...], w_ref[...],
                         preferred_element_type=jnp.float32)


def _matmul(x, w, br):
    rows, d = x.shape
    return pl.pallas_call(
        _matmul_body,
        grid=(rows // br,),
        in_specs=[
            pl.BlockSpec((br, d), lambda i: (i, 0)),
            pl.BlockSpec((d, w.shape[1]), lambda i: (0, 0)),
        ],
        out_specs=pl.BlockSpec((br, w.shape[1]), lambda i: (i, 0)),
        out_shape=jax.ShapeDtypeStruct((rows, w.shape[1]), jnp.float32),
    )(x, w)


def _final_body(p_ref, w_ref, o_ref):
    acc = p_ref[0] + p_ref[1]
    o_ref[...] = jnp.dot(acc, w_ref[...], preferred_element_type=jnp.float32)


def _final(p, w, br, rows):
    d = p.shape[2]
    return pl.pallas_call(
        _final_body,
        grid=(rows // br,),
        in_specs=[
            pl.BlockSpec((2, br, d), lambda i: (0, i, 0)),
            pl.BlockSpec((d, d), lambda i: (0, 0)),
        ],
        out_specs=pl.BlockSpec((br, d), lambda i: (i, 0)),
        out_shape=jax.ShapeDtypeStruct((rows, d), jnp.float32),
    )(p, w)


def _sc_edges(h_t, as_t, r_t, ar_t, idx3, wab, n_node, npad):
    d = h_t.shape[1]
    n_chunks = idx3.shape[1]
    rows_per_tile = npad // NS
    dch = d // L

    mesh = plsc.VectorSubcoreMesh(core_axis_name="c", subcore_axis_name="s",
                                  num_cores=NC, num_subcores=NS)

    @functools.partial(
        pl.kernel,
        out_type=jax.ShapeDtypeStruct((NC, npad, d), jnp.float32),
        mesh=mesh,
        compiler_params=pltpu.CompilerParams(needs_layout_passes=False),
        scratch_types=[
            pltpu.VMEM((3, 128), jnp.int32),
            pltpu.VMEM((3, 128), jnp.int32),
            pltpu.VMEM((C,), jnp.int32),
            pltpu.VMEM((C,), jnp.int32),
            pltpu.VMEM((C, 128), jnp.float32),
            pltpu.VMEM((C, 128), jnp.float32),
            pltpu.VMEM((C, 128), jnp.float32),
            pltpu.VMEM((C, 128), jnp.float32),
            pltpu.VMEM((C,), jnp.float32),
            pltpu.VMEM((L * L,), jnp.float32),
            pltpu.VMEM((144,), jnp.float32),
            pltpu.VMEM_SHARED((npad, d), jnp.float32),
            pltpu.SemaphoreType.DMA,
            pltpu.SemaphoreType.DMA,
            pltpu.SemaphoreType.DMA,
            pltpu.SemaphoreType.DMA,
        ],
    )
    def k(h_hbm, as_hbm, r_hbm, ar_hbm, idx3_hbm, wab_hbm,
          out_hbm, idx3_a, idx3_b, obj_a, obj_b, a1, a2, b1, b2, albuf, sbuf,
          wvec, acc, semA, semB, semI, semS):
        cid = lax.axis_index("c")
        sid = lax.axis_index("s")
        wid = cid * NS + sid

        def zfill(r, _):
            for j in range(dch):
                b1[r, pl.ds(j * L, L)] = jnp.zeros((L,), jnp.float32)
            return 0

        lax.fori_loop(0, C, zfill, 0)
        for part in range(rows_per_tile // C):
            pltpu.sync_copy(
                b1, acc.at[pl.ds(sid * rows_per_tile + part * C, C)])

        pltpu.sync_copy(wab_hbm, wvec)
        wv = [wvec[pl.ds(j * L, L)] for j in range(dch)]
        bias_v = jnp.full((L,), wvec[pl.ds(d, L)][0], jnp.float32)
        ev16 = lax.iota(jnp.int32, L) * L

        plsc.subcore_barrier()

        pltpu.async_copy(idx3_hbm.at[wid, 0], idx3_a, semI)
        pltpu.async_copy(idx3_hbm.at[wid, 0, 2, pl.ds(0, C)], obj_a, semI)

        def do_chunk(c, slot):
            i3 = idx3_a if slot == 0 else idx3_b
            obj = obj_a if slot == 0 else obj_b
            i3n = idx3_b if slot == 0 else idx3_a
            objn = obj_b if slot == 0 else obj_a

            @pl.when(c != 0)
            def _():
                pltpu.make_async_copy(
                    h_hbm.at[pl.ds(0, C)], b1, semS).wait()

            pltpu.make_async_copy(idx3_hbm.at[wid, 0], i3, semI).wait()
            pltpu.make_async_copy(
                idx3_hbm.at[wid, 0, 2, pl.ds(0, C)], obj, semI).wait()

            for i in range(C // L):
                v = obj[pl.ds(i * L, L)]
                obj[pl.ds(i * L, L)] = lax.rem(v, jnp.int32(n_node))

            i_sub = i3.at[0, pl.ds(0, C)]
            i_rel = i3.at[1, pl.ds(0, C)]
            cpA1 = pltpu.async_copy(as_hbm.at[i_sub], a1, semA)
            cpA2 = pltpu.async_copy(ar_hbm.at[i_rel], a2, semA)
            cpB1 = pltpu.async_copy(h_hbm.at[i_sub], b1, semB)
            cpB2 = pltpu.async_copy(r_hbm.at[i_rel], b2, semB)

            @pl.when(c + 1 < n_chunks)
            def _():
                pltpu.async_copy(idx3_hbm.at[wid, c + 1], i3n, semI)
                pltpu.async_copy(
                    idx3_hbm.at[wid, c + 1, 2, pl.ds(0, C)], objn, semI)

            cpA1.wait()
            cpA2.wait()

            def alpha_body(g, _):
                ebase = g * L
                for e16 in range(L):
                    i = ebase + e16
                    s = jnp.zeros((L,), jnp.float32)
                    for j in range(dch):
                        a = a1[i, pl.ds(j * L, L)] + a2[i, pl.ds(j * L, L)]
                        s = s + jnp.maximum(a, 0.0) * wv[j]
                    sbuf[pl.ds(e16 * L, L)] = s
                tsum = jnp.zeros((L,), jnp.float32)
                for c16 in range(L):
                    tsum = tsum + plsc.load_gather(sbuf, [ev16 + c16])
                alpha_v = 1.0 / (1.0 + jnp.exp(-(tsum + bias_v)))
                albuf[pl.ds(ebase, L)] = alpha_v
                return 0

            lax.fori_loop(0, C // L, alpha_body, 0)

            cpB1.wait()
            cpB2.wait()

            def msg_body(g, _):
                ebase = g * L
                alpha_v = albuf[pl.ds(ebase, L)]
                for e16 in range(L):
                    i = ebase + e16
                    av = jnp.full((L,), alpha_v[e16], jnp.float32)
                    for j in range(dch):
                        m = b1[i, pl.ds(j * L, L)] + b2[i, pl.ds(j * L, L)]
                        b1[i, pl.ds(j * L, L)] = av * m
                return 0

            lax.fori_loop(0, C // L, msg_body, 0)
            pltpu.async_copy(b1, acc.at[obj], semS, add=True)

        def pair_body(p, _):
            do_chunk(2 * p, 0)
            do_chunk(2 * p + 1, 1)
            return 0

        lax.fori_loop(0, n_chunks // 2, pair_body, 0)
        if n_chunks % 2:
            do_chunk(jnp.int32(n_chunks - 1), 0)

        pltpu.make_async_copy(h_hbm.at[pl.ds(0, C)], b1, semS).wait()
        plsc.subcore_barrier()
        pltpu.sync_copy(
            acc.at[pl.ds(sid * rows_per_tile, rows_per_tile)],
            out_hbm.at[cid, pl.ds(sid * rows_per_tile, rows_per_tile)])

    return k(h_t, as_t, r_t, ar_t, idx3, wab)


def kernel(hidden, edges, n_node, old_nodes_new_idx, rela_embed, Ws, Wr,
           w_alpha_w, w_alpha_b, W_h):
    n, d = hidden.shape
    v = rela_embed.shape[0]
    e = edges.shape[0]
    e_per_w = e // NW
    n_chunks = e_per_w // C

    cols = jnp.stack([edges[:, 4], edges[:, 2], edges[:, 5]])
    idx3 = jnp.transpose(
        cols.reshape(3, NW, n_chunks, C), (1, 2, 0, 3))
    idx3 = jnp.pad(idx3, ((0, 0), (0, 0), (0, 0), (0, 128 - C)))

    vpad = ((v + 399) // 400) * 400
    rela_p = jnp.pad(rela_embed, ((0, vpad - v), (0, 0)))

    as_t = _matmul(hidden, Ws, 400)
    ar_t = _matmul(rela_p, Wr, 400)

    wab = jnp.concatenate([
        w_alpha_w.reshape(d),
        w_alpha_b.reshape(1),
        jnp.zeros((15,), jnp.float32),
    ])

    npad = ((n + NS * 80 - 1) // (NS * 80)) * (NS * 80)
    partials = _sc_edges(hidden, as_t, rela_p, ar_t, idx3, wab, n, npad)
    return _final(partials, W_h, 400, n)
